# baseline (speedup 1.0000x reference)
import functools

import jax
import jax.numpy as jnp
import numpy as np
from jax import lax
from jax.experimental import pallas as pl
from jax.experimental.pallas import tpu as pltpu
from jax.experimental.pallas import tpu_sc as plsc

_G = 96
_LW = 128

_RP384 = np.array([
    2,3,5,6,8,9,13,14,17,18,22,23,24,25,30,31,33,34,38,39,41,42,45,47,48,51,
    53,54,56,57,60,61,65,66,68,70,73,75,77,79,80,81,85,87,89,91,94,95,96,99,
    101,103,104,105,110,111,112,114,116,117,121,123,125,127,128,129,133,135,
    137,139,140,141,144,146,148,150,152,154,156,157,161,163,164,166,169,171,
    174,175,178,179,180,181,185,186,189,190,194,195,196,198,200,203,204,206,
    209,211,214,215,216,218,221,223,224,227,229,230,232,233,237,239,241,242,
    244,246,250,251,254,255,256,257,262,263,266,267,268,271,273,274,276,279,
    282,283,284,287,289,291,293,294,296,298,300,303,304,307,308,310,313,315,
    317,318,320,323,325,326,330,331,332,334,338,339,341,342,344,346,349,350,
    353,355,357,358,362,363,366,367,368,369,373,374,377,379,381,382],
    dtype=np.int64)
_CQ384 = np.array([
    1,3,5,6,8,9,13,14,17,18,21,23,24,25,29,30,32,34,36,39,42,43,45,47,48,49,
    52,54,56,59,60,63,64,67,69,70,74,75,76,77,81,82,85,86,90,91,93,94,96,97,
    100,102,104,106,108,111,113,114,116,118,120,121,126,127,130,131,132,135,
    136,139,140,143,146,147,149,151,152,155,156,157,160,163,164,165,168,169,
    173,174,178,179,180,183,184,186,188,189,193,195,196,199,200,203,204,207,
    209,211,212,213,217,219,221,222,224,225,228,229,232,233,236,239,240,241,
    245,247,248,251,253,255,256,257,262,263,264,266,269,271,274,275,277,278,
    281,283,286,287,288,289,293,295,296,298,300,301,304,306,308,309,312,313,
    316,317,320,321,325,326,330,331,333,335,337,339,342,343,344,347,348,350,
    352,353,357,359,361,362,364,367,369,371,372,375,376,377,381,383],
    dtype=np.int64)


@functools.cache
def _make_sc_gather(R, S, NO, nwb, nob):
    info = plsc.get_sparse_core_info()
    nc, ns = info.num_cores, info.num_subcores
    nw = nc * ns
    rows_per = R // nw
    nchunks = rows_per // _G
    assert rows_per % _G == 0 and R % nw == 0 and NO % 16 == 0
    assert nchunks % 2 == 0
    mesh = plsc.VectorSubcoreMesh(core_axis_name="c", subcore_axis_name="s")

    @functools.partial(
        pl.kernel,
        mesh=mesh,
        compiler_params=pltpu.CompilerParams(
            needs_layout_passes=False, use_tc_tiling_on_sc=False),
        out_type=jax.ShapeDtypeStruct((S, _LW), jnp.float32),
        scratch_types=[
            pltpu.VMEM((nwb, _G), jnp.int32),
            pltpu.VMEM((nwb, _G), jnp.int32),
            pltpu.VMEM((NO,), jnp.int32),
            pltpu.VMEM((nwb, _G, _LW), jnp.float32),
            pltpu.VMEM((nwb, _G, _LW), jnp.float32),
            pltpu.VMEM((nob * _G, _LW), jnp.float32),
            pltpu.VMEM((nob * _G, _LW), jnp.float32),
            pltpu.SemaphoreType.DMA,
            pltpu.SemaphoreType.DMA,
            pltpu.SemaphoreType.DMA,
            pltpu.SemaphoreType.DMA,
        ],
    )
    def body(seg, sidx, cqm, out, idx_a, idx_b, cq_v,
             rows_a, rows_b, oseg_a, oseg_b, isem_a, isem_b, osem_a, osem_b):
        wid = lax.axis_index("s") * nc + lax.axis_index("c")
        chunk0 = wid * nchunks
        idxs = (idx_a, idx_b)
        rows = (rows_a, rows_b)
        osegs = (oseg_a, oseg_b)
        isems = (isem_a, isem_b)
        osems = (osem_a, osem_b)
        pltpu.sync_copy(cqm, cq_v)
        cqv = [cq_v[pl.ds(16 * j, 16)] for j in range(NO // 16)]

        def start_in(t, b):
            pltpu.sync_copy(sidx.at[chunk0 + t], idxs[b])
            for k in range(nwb):
                pltpu.async_copy(seg.at[idxs[b].at[k]], rows[b].at[k],
                                 isems[b])

        def wait_in(b):
            for k in range(nwb):
                pltpu.make_async_copy(seg.at[idxs[b].at[k]], rows[b].at[k],
                                      isems[b]).wait()

        nseg_chunk = nob * _G

        def out_dst(t):
            base = pl.multiple_of((chunk0 + t) * nseg_chunk, nseg_chunk)
            return out.at[pl.ds(base, nseg_chunk)]

        def start_out(t, b):
            pltpu.async_copy(osegs[b], out_dst(t), osems[b])

        def wait_out(t, b):
            pltpu.make_async_copy(osegs[b], out_dst(t), osems[b]).wait()

        start_in(0, 0)

        def pair(tt, carry):
            for b in range(2):
                t = tt * 2 + b
                wait_in(b)

                @pl.when(t + 1 < nchunks)
                def _():
                    start_in(t + 1, 1 - b)

                @pl.when(t >= 2)
                def _():
                    wait_out(t - 2, b)

                @plsc.parallel_loop(0, _G, 8, unroll=2)
                def _row(g0):
                    sl0 = 2 * g0
                    for g2 in range(8):
                        gs = jnp.full((16,), g0 + g2, jnp.int32)
                        for j in range(NO // 16):
                            ks = jnp.full((16,), j // 4, jnp.int32)
                            v = plsc.load_gather(rows[b], [ks, gs, cqv[j]])
                            osegs[b][sl0 + (j // 8) * 8 + g2,
                                     pl.ds(16 * (j % 8), 16)] = v

                start_out(t, b)
            return carry

        lax.fori_loop(0, nchunks // 2, pair, 0)
        for b in range(2):
            wait_out(nchunks - 2 + b, b)

    return body


def kernel(x):
    B, C, H, W = x.shape
    assert (H, W) == (384, 384)
    r_p, c_q = _RP384, _CQ384
    nr, no = r_p.shape[0], c_q.shape[0]
    planes = B * C
    nwb = W // _LW
    nob = (no + _LW - 1) // _LW
    nh = H // 8
    noh = nr // 8
    R = planes * nr
    S = planes * noh * nob * 8
    nchunks_total = R // _G

    seg = (x.reshape(B, C, nh, 8, nwb, _LW)
           .transpose(0, 1, 2, 4, 3, 5)
           .reshape(planes * nh * nwb * 8, _LW))

    r = np.arange(R, dtype=np.int64)
    p, i = r // nr, r % nr
    h = r_p[i]
    sid = (((p * nh + h // 8) * nwb)[None, :]
           + np.arange(nwb, dtype=np.int64)[:, None]) * 8 + (h % 8)[None, :]
    sid = (sid.reshape(nwb, nchunks_total, _G).transpose(1, 0, 2)
           .astype(np.int32))

    assert np.all(c_q // _LW == np.arange(no) // (no // nwb))
    cqm = (c_q % _LW).astype(np.int32)

    y = _make_sc_gather(R, S, no, nwb, nob)(
        seg, jnp.asarray(sid), jnp.asarray(cqm))
    y = (y.reshape(B, C, noh, nob, 8, _LW)
         .transpose(0, 1, 2, 4, 3, 5)
         .reshape(B, C, nr, nob * _LW))
    return y[:, :, :, :no]


# speedup vs baseline: 1.0458x; 1.0458x over previous
import functools

import jax
import jax.numpy as jnp
import numpy as np
from jax import lax
from jax.experimental import pallas as pl
from jax.experimental.pallas import tpu as pltpu
from jax.experimental.pallas import tpu_sc as plsc

_G = 96
_LW = 128

_RP384 = np.array([
    2,3,5,6,8,9,13,14,17,18,22,23,24,25,30,31,33,34,38,39,41,42,45,47,48,51,
    53,54,56,57,60,61,65,66,68,70,73,75,77,79,80,81,85,87,89,91,94,95,96,99,
    101,103,104,105,110,111,112,114,116,117,121,123,125,127,128,129,133,135,
    137,139,140,141,144,146,148,150,152,154,156,157,161,163,164,166,169,171,
    174,175,178,179,180,181,185,186,189,190,194,195,196,198,200,203,204,206,
    209,211,214,215,216,218,221,223,224,227,229,230,232,233,237,239,241,242,
    244,246,250,251,254,255,256,257,262,263,266,267,268,271,273,274,276,279,
    282,283,284,287,289,291,293,294,296,298,300,303,304,307,308,310,313,315,
    317,318,320,323,325,326,330,331,332,334,338,339,341,342,344,346,349,350,
    353,355,357,358,362,363,366,367,368,369,373,374,377,379,381,382],
    dtype=np.int64)
_CQ384 = np.array([
    1,3,5,6,8,9,13,14,17,18,21,23,24,25,29,30,32,34,36,39,42,43,45,47,48,49,
    52,54,56,59,60,63,64,67,69,70,74,75,76,77,81,82,85,86,90,91,93,94,96,97,
    100,102,104,106,108,111,113,114,116,118,120,121,126,127,130,131,132,135,
    136,139,140,143,146,147,149,151,152,155,156,157,160,163,164,165,168,169,
    173,174,178,179,180,183,184,186,188,189,193,195,196,199,200,203,204,207,
    209,211,212,213,217,219,221,222,224,225,228,229,232,233,236,239,240,241,
    245,247,248,251,253,255,256,257,262,263,264,266,269,271,274,275,277,278,
    281,283,286,287,288,289,293,295,296,298,300,301,304,306,308,309,312,313,
    316,317,320,321,325,326,330,331,333,335,337,339,342,343,344,347,348,350,
    352,353,357,359,361,362,364,367,369,371,372,375,376,377,381,383],
    dtype=np.int64)


@functools.cache
def _make_sc_gather(R, S, NO, nwb, nob):
    info = plsc.get_sparse_core_info()
    nc, ns = info.num_cores, info.num_subcores
    nw = nc * ns
    rows_per = R // nw
    nchunks = rows_per // _G
    assert rows_per % _G == 0 and R % nw == 0 and NO % 16 == 0
    assert nchunks % 2 == 0
    mesh = plsc.VectorSubcoreMesh(core_axis_name="c", subcore_axis_name="s")

    @functools.partial(
        pl.kernel,
        mesh=mesh,
        compiler_params=pltpu.CompilerParams(
            needs_layout_passes=False, use_tc_tiling_on_sc=False),
        out_type=jax.ShapeDtypeStruct((S, _LW), jnp.float32),
        scratch_types=[
            pltpu.VMEM((nwb, _G), jnp.int32),
            pltpu.VMEM((nwb, _G), jnp.int32),
            pltpu.VMEM((nob, _G), jnp.int32),
            pltpu.VMEM((nob, _G), jnp.int32),
            pltpu.VMEM((NO,), jnp.int32),
            pltpu.VMEM((nwb, _G, _LW), jnp.float32),
            pltpu.VMEM((nwb, _G, _LW), jnp.float32),
            pltpu.VMEM((nob, _G, _LW), jnp.float32),
            pltpu.VMEM((nob, _G, _LW), jnp.float32),
            pltpu.SemaphoreType.DMA,
            pltpu.SemaphoreType.DMA,
            pltpu.SemaphoreType.DMA,
            pltpu.SemaphoreType.DMA,
        ],
    )
    def body(seg, sidx, soidx, cqm, out, idx_a, idx_b, oidx_a, oidx_b, cq_v,
             rows_a, rows_b, oseg_a, oseg_b, isem_a, isem_b, osem_a, osem_b):
        wid = lax.axis_index("s") * nc + lax.axis_index("c")
        chunk0 = wid * nchunks
        idxs = (idx_a, idx_b)
        oidxs = (oidx_a, oidx_b)
        rows = (rows_a, rows_b)
        osegs = (oseg_a, oseg_b)
        isems = (isem_a, isem_b)
        osems = (osem_a, osem_b)
        pltpu.sync_copy(cqm, cq_v)
        cqv = [cq_v[pl.ds(16 * j, 16)] for j in range(NO // 16)]

        def start_in(t, b):
            pltpu.sync_copy(sidx.at[chunk0 + t], idxs[b])
            for k in range(nwb):
                pltpu.async_copy(seg.at[idxs[b].at[k]], rows[b].at[k],
                                 isems[b])

        def wait_in(b):
            for k in range(nwb):
                pltpu.make_async_copy(seg.at[idxs[b].at[k]], rows[b].at[k],
                                      isems[b]).wait()

        def start_out(b):
            for k in range(nob):
                pltpu.async_copy(osegs[b].at[k], out.at[oidxs[b].at[k]],
                                 osems[b])

        def wait_out(b):
            for k in range(nob):
                pltpu.make_async_copy(osegs[b].at[k], out.at[oidxs[b].at[k]],
                                      osems[b]).wait()

        start_in(0, 0)

        def pair(tt, carry):
            for b in range(2):
                t = tt * 2 + b
                wait_in(b)

                @pl.when(t + 1 < nchunks)
                def _():
                    start_in(t + 1, 1 - b)

                @pl.when(t >= 2)
                def _():
                    wait_out(b)

                pltpu.sync_copy(soidx.at[chunk0 + t], oidxs[b])

                @plsc.parallel_loop(0, _G, 1, unroll=2)
                def _row(g):
                    gs = jnp.full((16,), g, jnp.int32)
                    for j in range(NO // 16):
                        ks = jnp.full((16,), j // 4, jnp.int32)
                        v = plsc.load_gather(rows[b], [ks, gs, cqv[j]])
                        osegs[b][j // 8, g, pl.ds(16 * (j % 8), 16)] = v

                start_out(b)
            return carry

        lax.fori_loop(0, nchunks // 2, pair, 0)
        for b in range(2):
            wait_out(b)

    return body


def kernel(x):
    B, C, H, W = x.shape
    assert (H, W) == (384, 384)
    r_p, c_q = _RP384, _CQ384
    nr, no = r_p.shape[0], c_q.shape[0]
    planes = B * C
    nwb = W // _LW
    nob = (no + _LW - 1) // _LW
    nh = H // 8
    noh = nr // 8
    R = planes * nr
    S = planes * noh * nob * 8
    nchunks_total = R // _G

    seg = (x.reshape(B, C, nh, 8, nwb, _LW)
           .transpose(0, 1, 2, 4, 3, 5)
           .reshape(planes * nh * nwb * 8, _LW))

    r = np.arange(R, dtype=np.int64)
    p, i = r // nr, r % nr
    h = r_p[i]
    sid = (((p * nh + h // 8) * nwb)[None, :]
           + np.arange(nwb, dtype=np.int64)[:, None]) * 8 + (h % 8)[None, :]
    sid = (sid.reshape(nwb, nchunks_total, _G).transpose(1, 0, 2)
           .astype(np.int32))

    soid = (((p * noh + i // 8) * nob)[None, :]
            + np.arange(nob, dtype=np.int64)[:, None]) * 8 + (i % 8)[None, :]
    soid = (soid.reshape(nob, nchunks_total, _G).transpose(1, 0, 2)
            .astype(np.int32))

    assert np.all(c_q // _LW == np.arange(no) // (no // nwb))
    cqm = (c_q % _LW).astype(np.int32)

    y = _make_sc_gather(R, S, no, nwb, nob)(
        seg, jnp.asarray(sid), jnp.asarray(soid), jnp.asarray(cqm))
    y = (y.reshape(B, C, noh, nob, 8, _LW)
         .transpose(0, 1, 2, 4, 3, 5)
         .reshape(B, C, nr, nob * _LW))
    return y[:, :, :, :no]
